# baseline (speedup 1.0000x reference)
import functools

import jax
import jax.numpy as jnp
from jax import lax
from jax.experimental import pallas as pl
from jax.experimental.pallas import tpu as pltpu
from jax.experimental.pallas import tpu_sc as plsc

_NEG = -1e30


def _sc_gather(table, idx3, n_out_rows):
    nw, chunks, chunk = idx3.shape
    d = table.shape[1]
    rows_per_w = chunks * chunk
    mesh = plsc.VectorSubcoreMesh(core_axis_name="c", subcore_axis_name="s")

    @functools.partial(
        pl.kernel,
        mesh=mesh,
        out_type=jax.ShapeDtypeStruct((n_out_rows, d), table.dtype),
        scratch_types=[
            pltpu.VMEM((chunks, chunk), jnp.int32),
            pltpu.VMEM((chunk, d), table.dtype),
            pltpu.VMEM((chunk, d), table.dtype),
            pltpu.SemaphoreType.DMA,
            pltpu.SemaphoreType.DMA,
        ],
    )
    def k(table_hbm, idx_hbm, out_hbm, idx_v, rows_v0, rows_v1, sem0, sem1):
        wid = lax.axis_index("s") * 2 + lax.axis_index("c")
        pltpu.sync_copy(idx_hbm.at[wid], idx_v)
        base = wid * rows_per_w
        bufs = (rows_v0, rows_v1)
        sems = (sem0, sem1)
        pend = [pltpu.async_copy(table_hbm.at[idx_v.at[0]], rows_v0, sem0),
                None]
        for c in range(chunks):
            if c + 1 < chunks:
                nb = (c + 1) % 2
                pend[nb] = pltpu.async_copy(
                    table_hbm.at[idx_v.at[c + 1]], bufs[nb], sems[nb])
            pend[c % 2].wait()
            pltpu.sync_copy(bufs[c % 2],
                            out_hbm.at[pl.ds(base + c * chunk, chunk)])

    return k(table, idx3)


def _sc_dispatch(x, s0_3, s1_3, n_out_rows):
    nw, chunks, tpc = s0_3.shape
    d = x.shape[1]
    tpw = chunks * tpc
    mesh = plsc.VectorSubcoreMesh(core_axis_name="c", subcore_axis_name="s")

    @functools.partial(
        pl.kernel,
        mesh=mesh,
        out_type=jax.ShapeDtypeStruct((n_out_rows, d), x.dtype),
        scratch_types=[
            pltpu.VMEM((chunks, tpc), jnp.int32),
            pltpu.VMEM((chunks, tpc), jnp.int32),
            pltpu.VMEM((tpc, d), x.dtype),
            pltpu.VMEM((tpc, d), x.dtype),
            pltpu.SemaphoreType.DMA,
            pltpu.SemaphoreType.DMA,
            pltpu.SemaphoreType.DMA,
            pltpu.SemaphoreType.DMA,
        ],
    )
    def k(x_hbm, s0_hbm, s1_hbm, out_hbm, s0_v, s1_v, buf0, buf1,
          sg0, sg1, ss0, ss1):
        wid = lax.axis_index("s") * 2 + lax.axis_index("c")
        pltpu.sync_copy(s0_hbm.at[wid], s0_v)
        pltpu.sync_copy(s1_hbm.at[wid], s1_v)
        base = wid * tpw
        bufs = (buf0, buf1)
        sg = (sg0, sg1)
        ss = (ss0, ss1)
        pend_g = [pltpu.async_copy(x_hbm.at[pl.ds(base, tpc)], buf0, sg0),
                  None]
        pend_s = [None, None]
        for c in range(chunks):
            if c + 1 < chunks:
                nb = (c + 1) % 2
                if pend_s[nb] is not None:
                    for h in pend_s[nb]:
                        h.wait()
                    pend_s[nb] = None
                pend_g[nb] = pltpu.async_copy(
                    x_hbm.at[pl.ds(base + (c + 1) * tpc, tpc)], bufs[nb],
                    sg[nb])
            cb = c % 2
            pend_g[cb].wait()
            pend_s[cb] = (
                pltpu.async_copy(bufs[cb], out_hbm.at[s0_v.at[c]], ss[cb]),
                pltpu.async_copy(bufs[cb], out_hbm.at[s1_v.at[c]], ss[cb]),
            )
        for p in pend_s:
            if p is not None:
                for h in p:
                    h.wait()

    return k(x, s0_3, s1_3)


def kernel(x, Wr, router_bias, Wg, Wu, Wd, Wsg, Wsu, Wsd):
    Bb, Tt, D = x.shape
    N = Bb * Tt
    E, _, H = Wg.shape
    S, _, HS = Wsg.shape
    K = 2
    M = 128
    NB = (K * N) // M + E
    NSLOT = NB * M
    TB = 256
    NW = 32

    flat = x.reshape(N, D)
    f32 = jnp.float32

    wrt = jnp.zeros((D, 128), f32).at[:, :E].set(Wr.T)
    biasp = jnp.full((1, 128), _NEG, f32).at[0, :E].set(router_bias)

    def router_body(x_ref, wrt_ref, bias_ref, idx_ref, w_ref, m_ref):
        xb = x_ref[...]
        logits = jnp.dot(xb, wrt_ref[...], preferred_element_type=f32)
        lane = lax.broadcasted_iota(jnp.int32, logits.shape, 1)
        valid = lane < E
        lm = jnp.where(valid, logits, _NEG)
        mx = jnp.max(lm, axis=1, keepdims=True)
        p = jnp.exp(lm - mx)
        scores = p / jnp.sum(p, axis=1, keepdims=True)
        sel = lm + bias_ref[...]
        m0 = jnp.max(sel, axis=1, keepdims=True)
        i0 = jnp.min(jnp.where(sel == m0, lane, 1000), axis=1, keepdims=True)
        pick0 = lane == i0
        sel2 = jnp.where(pick0, _NEG, sel)
        m1 = jnp.max(sel2, axis=1, keepdims=True)
        i1 = jnp.min(jnp.where(sel2 == m1, lane, 1000), axis=1, keepdims=True)
        pick1 = lane == i1
        w0 = jnp.sum(jnp.where(pick0, scores, 0.0), axis=1, keepdims=True)
        w1 = jnp.sum(jnp.where(pick1, scores, 0.0), axis=1, keepdims=True)
        tot = w0 + w1
        idx_ref[...] = jnp.where(lane == 0, i0, jnp.where(lane == 1, i1, 0))
        w_ref[...] = jnp.where(
            lane == 0, w0 / tot, jnp.where(lane == 1, w1 / tot, 0.0))
        m_ref[...] = jnp.where(pick0 | pick1, 1.0, 0.0)

    idx_out, w_out, m_out = pl.pallas_call(
        router_body,
        grid=(N // TB,),
        in_specs=[
            pl.BlockSpec((TB, D), lambda i: (i, 0)),
            pl.BlockSpec((D, 128), lambda i: (0, 0)),
            pl.BlockSpec((1, 128), lambda i: (0, 0)),
        ],
        out_specs=[
            pl.BlockSpec((TB, 128), lambda i: (i, 0)),
            pl.BlockSpec((TB, 128), lambda i: (i, 0)),
            pl.BlockSpec((TB, 128), lambda i: (i, 0)),
        ],
        out_shape=[
            jax.ShapeDtypeStruct((N, 128), jnp.int32),
            jax.ShapeDtypeStruct((N, 128), f32),
            jax.ShapeDtypeStruct((N, 128), f32),
        ],
    )(flat, wrt, biasp)

    TBS = 1024
    CH = 256
    HC = HS // CH
    NJ = S * HC

    def shared_body(x_ref, wsg_ref, wsu_ref, wsd_ref, o_ref):
        j = pl.program_id(1)
        bf = jnp.bfloat16
        xb = x_ref[...].astype(bf)
        g = jnp.dot(xb, wsg_ref[0].astype(bf), preferred_element_type=f32)
        u = jnp.dot(xb, wsu_ref[0].astype(bf), preferred_element_type=f32)
        h = (g * jax.nn.sigmoid(g) * u).astype(bf)
        part = jnp.dot(h, wsd_ref[0].astype(bf), preferred_element_type=f32)

        @pl.when(j == 0)
        def _():
            o_ref[...] = part

        @pl.when(j > 0)
        def _():
            o_ref[...] += part

    shared_out = pl.pallas_call(
        shared_body,
        grid=(N // TBS, NJ),
        in_specs=[
            pl.BlockSpec((TBS, D), lambda i, j: (i, 0)),
            pl.BlockSpec((1, D, CH), lambda i, j, _h=HC: (j // _h, 0, j % _h)),
            pl.BlockSpec((1, D, CH), lambda i, j, _h=HC: (j // _h, 0, j % _h)),
            pl.BlockSpec((1, CH, D), lambda i, j, _h=HC: (j // _h, j % _h, 0)),
        ],
        out_specs=pl.BlockSpec((TBS, D), lambda i, j: (i, 0)),
        out_shape=jax.ShapeDtypeStruct((N, D), f32),
    )(flat, Wsg, Wsu, Wsd)


    tri = jnp.triu(jnp.ones((128, 128), f32), k=1)

    def dispatch_body(m_ref, idx_ref, tri_ref, slot_ref, meta_ref):
        m = m_ref[...]
        lane = lax.broadcasted_iota(jnp.int32, m.shape, 1)
        inc = m
        sh = 1
        while sh < N:
            inc = inc + jnp.concatenate(
                [jnp.zeros((sh, 128), f32), inc[:N - sh]], axis=0)
            sh *= 2
        excl = inc - m
        counts = inc[N - 1:N, :]
        bc = jnp.floor((counts + (M - 1)) * (1.0 / M))
        blk_start = jnp.dot(bc, tri_ref[...], preferred_element_type=f32)
        tot = excl + blk_start * M
        i0 = idx_ref[:, 0:1]
        i1 = idx_ref[:, 1:2]
        s0 = jnp.sum(jnp.where(lane == i0, tot, 0.0), axis=1, keepdims=True)
        s1 = jnp.sum(jnp.where(lane == i1, tot, 0.0), axis=1, keepdims=True)
        slot_ref[...] = jnp.where(
            lane == 0, s0, jnp.where(lane == 1, s1, 0.0)).astype(jnp.int32)
        srow = lax.broadcasted_iota(jnp.int32, (8, 128), 0)
        meta_ref[...] = jnp.where(
            srow == 0, jnp.broadcast_to(blk_start, (8, 128)),
            jnp.where(srow == 1, jnp.broadcast_to(bc, (8, 128)),
                      0.0)).astype(jnp.int32)

    slot_out, meta = pl.pallas_call(
        dispatch_body,
        grid=(1,),
        in_specs=[
            pl.BlockSpec((N, 128), lambda i: (0, 0)),
            pl.BlockSpec((N, 128), lambda i: (0, 0)),
            pl.BlockSpec((128, 128), lambda i: (0, 0)),
        ],
        out_specs=[
            pl.BlockSpec((N, 128), lambda i: (0, 0)),
            pl.BlockSpec((8, 128), lambda i: (0, 0)),
        ],
        out_shape=[
            jax.ShapeDtypeStruct((N, 128), jnp.int32),
            jax.ShapeDtypeStruct((8, 128), jnp.int32),
        ],
    )(m_out, idx_out, tri)

    bs8 = meta[0, :E]
    bc8 = meta[1, :E]
    num_active = jnp.sum(bc8).astype(jnp.int32)
    na_arr = num_active.reshape(1)
    barange = jnp.arange(NB, dtype=jnp.int32)
    owner = jnp.sum(
        (bs8[None, :] <= barange[:, None]).astype(jnp.int32), axis=1) - 1
    last_e = jnp.max(jnp.where(bc8 > 0, jnp.arange(E), 0)).astype(jnp.int32)
    block_expert = jnp.where(barange < num_active, owner, last_e).astype(
        jnp.int32)
    s0 = slot_out[:, 0]
    s1 = slot_out[:, 1]

    TPC = 8
    xd = _sc_dispatch(flat, s0.reshape(NW, N // (NW * TPC), TPC),
                      s1.reshape(NW, N // (NW * TPC), TPC), NSLOT)

    def grouped_body(be_ref, na_ref, xd_ref, wg_ref, wu_ref, wd_ref, yw_ref):
        b = pl.program_id(0)

        @pl.when(b < na_ref[0])
        def _():
            bf = jnp.bfloat16
            xb = xd_ref[...].astype(bf)
            g = jnp.dot(xb, wg_ref[0].astype(bf), preferred_element_type=f32)
            u = jnp.dot(xb, wu_ref[0].astype(bf), preferred_element_type=f32)
            h = (g * jax.nn.sigmoid(g) * u).astype(bf)
            yw_ref[...] = jnp.dot(h, wd_ref[0].astype(bf),
                                  preferred_element_type=f32).astype(bf)

    grid_spec = pltpu.PrefetchScalarGridSpec(
        num_scalar_prefetch=2,
        grid=(NB,),
        in_specs=[
            pl.BlockSpec((M, D), lambda b, be, na: (b, 0)),
            pl.BlockSpec((1, D, H), lambda b, be, na: (be[b], 0, 0)),
            pl.BlockSpec((1, D, H), lambda b, be, na: (be[b], 0, 0)),
            pl.BlockSpec((1, H, D), lambda b, be, na: (be[b], 0, 0)),
        ],
        out_specs=pl.BlockSpec((M, D), lambda b, be, na: (b, 0)),
    )
    yw = pl.pallas_call(
        grouped_body,
        grid_spec=grid_spec,
        out_shape=jax.ShapeDtypeStruct((NSLOT, D), jnp.bfloat16),
    )(block_expert, na_arr, xd, Wg, Wu, Wd)

    pos_all = jnp.concatenate([s0, s1]).astype(jnp.int32)
    yw32 = jax.lax.bitcast_convert_type(
        yw.reshape(NSLOT, D // 2, 2), jnp.int32)
    yg32 = _sc_gather(yw32, pos_all.reshape(NW, (N * K) // (NW * 16), 16),
                      N * K)
    yg = jax.lax.bitcast_convert_type(yg32, jnp.bfloat16).reshape(N * K, D)

    def combine_body(s_ref, y0_ref, y1_ref, w_ref, o_ref):
        o_ref[...] = (s_ref[...]
                      + y0_ref[...].astype(f32) * w_ref[:, 0:1]
                      + y1_ref[...].astype(f32) * w_ref[:, 1:2])

    nblk = N // TB
    out = pl.pallas_call(
        combine_body,
        grid=(nblk,),
        in_specs=[
            pl.BlockSpec((TB, D), lambda i: (i, 0)),
            pl.BlockSpec((TB, D), lambda i: (i, 0)),
            pl.BlockSpec((TB, D), lambda i, _n=nblk: (i + _n, 0)),
            pl.BlockSpec((TB, 128), lambda i: (i, 0)),
        ],
        out_specs=pl.BlockSpec((TB, D), lambda i: (i, 0)),
        out_shape=jax.ShapeDtypeStruct((N, D), f32),
    )(shared_out, yg, yg, w_out)

    return out.reshape(Bb, Tt, D)


# speedup vs baseline: 2.4662x; 2.4662x over previous
import functools

import jax
import jax.numpy as jnp
from jax import lax
from jax.experimental import pallas as pl
from jax.experimental.pallas import tpu as pltpu
from jax.experimental.pallas import tpu_sc as plsc

_NEG = -1e30


def _sc_gather(table, idx3, n_out_rows, nbuf=2):
    nw, chunks, chunk = idx3.shape
    d = table.shape[1]
    rows_per_w = chunks * chunk
    mesh = plsc.VectorSubcoreMesh(core_axis_name="c", subcore_axis_name="s")

    @functools.partial(
        pl.kernel,
        mesh=mesh,
        out_type=jax.ShapeDtypeStruct((n_out_rows, d), table.dtype),
        scratch_types=[
            pltpu.VMEM((chunks, chunk), jnp.int32),
            pltpu.VMEM((chunk, d), table.dtype),
            pltpu.VMEM((chunk, d), table.dtype),
            pltpu.SemaphoreType.DMA,
            pltpu.SemaphoreType.DMA,
        ],
    )
    def k(table_hbm, idx_hbm, out_hbm, idx_v, rows_v0, rows_v1, sem0, sem1):
        wid = lax.axis_index("s") * 2 + lax.axis_index("c")
        pltpu.sync_copy(idx_hbm.at[wid], idx_v)
        base = wid * rows_per_w
        if nbuf == 1:
            for c in range(chunks):
                pltpu.async_copy(
                    table_hbm.at[idx_v.at[c]], rows_v0, sem0).wait()
                pltpu.sync_copy(rows_v0,
                                out_hbm.at[pl.ds(base + c * chunk, chunk)])
            return
        bufs = (rows_v0, rows_v1)
        sems = (sem0, sem1)
        pend = [pltpu.async_copy(table_hbm.at[idx_v.at[0]], rows_v0, sem0),
                None]
        for c in range(chunks):
            if c + 1 < chunks:
                nb = (c + 1) % 2
                pend[nb] = pltpu.async_copy(
                    table_hbm.at[idx_v.at[c + 1]], bufs[nb], sems[nb])
            pend[c % 2].wait()
            pltpu.sync_copy(bufs[c % 2],
                            out_hbm.at[pl.ds(base + c * chunk, chunk)])

    return k(table, idx3)


def _sc_dispatch(x, s0_3, s1_3, n_out_rows):
    nw, chunks, tpc = s0_3.shape
    d = x.shape[1]
    tpw = chunks * tpc
    mesh = plsc.VectorSubcoreMesh(core_axis_name="c", subcore_axis_name="s")

    @functools.partial(
        pl.kernel,
        mesh=mesh,
        out_type=jax.ShapeDtypeStruct((n_out_rows, d), x.dtype),
        scratch_types=[
            pltpu.VMEM((chunks, tpc), jnp.int32),
            pltpu.VMEM((chunks, tpc), jnp.int32),
            pltpu.VMEM((tpc, d), x.dtype),
            pltpu.VMEM((tpc, d), x.dtype),
            pltpu.SemaphoreType.DMA,
            pltpu.SemaphoreType.DMA,
            pltpu.SemaphoreType.DMA,
            pltpu.SemaphoreType.DMA,
        ],
    )
    def k(x_hbm, s0_hbm, s1_hbm, out_hbm, s0_v, s1_v, buf0, buf1,
          sg0, sg1, ss0, ss1):
        wid = lax.axis_index("s") * 2 + lax.axis_index("c")
        pltpu.sync_copy(s0_hbm.at[wid], s0_v)
        pltpu.sync_copy(s1_hbm.at[wid], s1_v)
        base = wid * tpw
        bufs = (buf0, buf1)
        sg = (sg0, sg1)
        ss = (ss0, ss1)
        pend_g = [pltpu.async_copy(x_hbm.at[pl.ds(base, tpc)], buf0, sg0),
                  None]
        pend_s = [None, None]
        for c in range(chunks):
            if c + 1 < chunks:
                nb = (c + 1) % 2
                if pend_s[nb] is not None:
                    for h in pend_s[nb]:
                        h.wait()
                    pend_s[nb] = None
                pend_g[nb] = pltpu.async_copy(
                    x_hbm.at[pl.ds(base + (c + 1) * tpc, tpc)], bufs[nb],
                    sg[nb])
            cb = c % 2
            pend_g[cb].wait()
            pend_s[cb] = (
                pltpu.async_copy(bufs[cb], out_hbm.at[s0_v.at[c]], ss[cb]),
                pltpu.async_copy(bufs[cb], out_hbm.at[s1_v.at[c]], ss[cb]),
            )
        for p in pend_s:
            if p is not None:
                for h in p:
                    h.wait()

    return k(x, s0_3, s1_3)


def kernel(x, Wr, router_bias, Wg, Wu, Wd, Wsg, Wsu, Wsd):
    Bb, Tt, D = x.shape
    N = Bb * Tt
    E, _, H = Wg.shape
    S, _, HS = Wsg.shape
    K = 2
    M = 128
    NB = (K * N) // M + E
    NSLOT = NB * M
    TB = 256
    NW = 32

    flat = x.reshape(N, D)
    f32 = jnp.float32

    wrt = jnp.zeros((D, 128), f32).at[:, :E].set(Wr.T)
    biasp = jnp.full((1, 128), _NEG, f32).at[0, :E].set(router_bias)

    def router_body(x_ref, wrt_ref, bias_ref, idx_ref, w_ref, m_ref):
        xb = x_ref[...]
        logits = jnp.dot(xb, wrt_ref[...], preferred_element_type=f32)
        lane = lax.broadcasted_iota(jnp.int32, logits.shape, 1)
        valid = lane < E
        lm = jnp.where(valid, logits, _NEG)
        mx = jnp.max(lm, axis=1, keepdims=True)
        p = jnp.exp(lm - mx)
        scores = p / jnp.sum(p, axis=1, keepdims=True)
        sel = lm + bias_ref[...]
        m0 = jnp.max(sel, axis=1, keepdims=True)
        i0 = jnp.min(jnp.where(sel == m0, lane, 1000), axis=1, keepdims=True)
        pick0 = lane == i0
        sel2 = jnp.where(pick0, _NEG, sel)
        m1 = jnp.max(sel2, axis=1, keepdims=True)
        i1 = jnp.min(jnp.where(sel2 == m1, lane, 1000), axis=1, keepdims=True)
        pick1 = lane == i1
        w0 = jnp.sum(jnp.where(pick0, scores, 0.0), axis=1, keepdims=True)
        w1 = jnp.sum(jnp.where(pick1, scores, 0.0), axis=1, keepdims=True)
        tot = w0 + w1
        idx_ref[...] = jnp.where(lane == 0, i0, jnp.where(lane == 1, i1, 0))
        w_ref[...] = jnp.where(
            lane == 0, w0 / tot, jnp.where(lane == 1, w1 / tot, 0.0))
        m_ref[...] = jnp.where(pick0 | pick1, 1.0, 0.0)

    idx_out, w_out, m_out = pl.pallas_call(
        router_body,
        grid=(N // TB,),
        in_specs=[
            pl.BlockSpec((TB, D), lambda i: (i, 0)),
            pl.BlockSpec((D, 128), lambda i: (0, 0)),
            pl.BlockSpec((1, 128), lambda i: (0, 0)),
        ],
        out_specs=[
            pl.BlockSpec((TB, 128), lambda i: (i, 0)),
            pl.BlockSpec((TB, 128), lambda i: (i, 0)),
            pl.BlockSpec((TB, 128), lambda i: (i, 0)),
        ],
        out_shape=[
            jax.ShapeDtypeStruct((N, 128), jnp.int32),
            jax.ShapeDtypeStruct((N, 128), f32),
            jax.ShapeDtypeStruct((N, 128), f32),
        ],
    )(flat, wrt, biasp)

    TBS = 1024
    CH = 256
    HC = HS // CH
    NJ = S * HC

    def shared_body(x_ref, wsg_ref, wsu_ref, wsd_ref, o_ref):
        j = pl.program_id(1)
        bf = jnp.bfloat16
        xb = x_ref[...].astype(bf)
        g = jnp.dot(xb, wsg_ref[0].astype(bf), preferred_element_type=f32)
        u = jnp.dot(xb, wsu_ref[0].astype(bf), preferred_element_type=f32)
        h = (g * jax.nn.sigmoid(g) * u).astype(bf)
        part = jnp.dot(h, wsd_ref[0].astype(bf), preferred_element_type=f32)

        @pl.when(j == 0)
        def _():
            o_ref[...] = part

        @pl.when(j > 0)
        def _():
            o_ref[...] += part

    shared_out = pl.pallas_call(
        shared_body,
        grid=(N // TBS, NJ),
        in_specs=[
            pl.BlockSpec((TBS, D), lambda i, j: (i, 0)),
            pl.BlockSpec((1, D, CH), lambda i, j, _h=HC: (j // _h, 0, j % _h)),
            pl.BlockSpec((1, D, CH), lambda i, j, _h=HC: (j // _h, 0, j % _h)),
            pl.BlockSpec((1, CH, D), lambda i, j, _h=HC: (j // _h, j % _h, 0)),
        ],
        out_specs=pl.BlockSpec((TBS, D), lambda i, j: (i, 0)),
        out_shape=jax.ShapeDtypeStruct((N, D), f32),
    )(flat, Wsg, Wsu, Wsd)


    tri = jnp.triu(jnp.ones((128, 128), f32), k=1)

    def dispatch_body(m_ref, idx_ref, tri_ref, slot_ref, meta_ref):
        m = m_ref[...]
        lane = lax.broadcasted_iota(jnp.int32, m.shape, 1)
        inc = m
        sh = 1
        while sh < N:
            inc = inc + jnp.concatenate(
                [jnp.zeros((sh, 128), f32), inc[:N - sh]], axis=0)
            sh *= 2
        excl = inc - m
        counts = inc[N - 1:N, :]
        bc = jnp.floor((counts + (M - 1)) * (1.0 / M))
        blk_start = jnp.dot(bc, tri_ref[...], preferred_element_type=f32)
        tot = excl + blk_start * M
        i0 = idx_ref[:, 0:1]
        i1 = idx_ref[:, 1:2]
        s0 = jnp.sum(jnp.where(lane == i0, tot, 0.0), axis=1, keepdims=True)
        s1 = jnp.sum(jnp.where(lane == i1, tot, 0.0), axis=1, keepdims=True)
        slot_ref[...] = jnp.where(
            lane == 0, s0, jnp.where(lane == 1, s1, 0.0)).astype(jnp.int32)
        srow = lax.broadcasted_iota(jnp.int32, (8, 128), 0)
        meta_ref[...] = jnp.where(
            srow == 0, jnp.broadcast_to(blk_start, (8, 128)),
            jnp.where(srow == 1, jnp.broadcast_to(bc, (8, 128)),
                      0.0)).astype(jnp.int32)

    slot_out, meta = pl.pallas_call(
        dispatch_body,
        grid=(1,),
        in_specs=[
            pl.BlockSpec((N, 128), lambda i: (0, 0)),
            pl.BlockSpec((N, 128), lambda i: (0, 0)),
            pl.BlockSpec((128, 128), lambda i: (0, 0)),
        ],
        out_specs=[
            pl.BlockSpec((N, 128), lambda i: (0, 0)),
            pl.BlockSpec((8, 128), lambda i: (0, 0)),
        ],
        out_shape=[
            jax.ShapeDtypeStruct((N, 128), jnp.int32),
            jax.ShapeDtypeStruct((8, 128), jnp.int32),
        ],
    )(m_out, idx_out, tri)

    bs8 = meta[0, :E]
    bc8 = meta[1, :E]
    num_active = jnp.sum(bc8).astype(jnp.int32)
    na_arr = num_active.reshape(1)
    barange = jnp.arange(NB, dtype=jnp.int32)
    owner = jnp.sum(
        (bs8[None, :] <= barange[:, None]).astype(jnp.int32), axis=1) - 1
    last_e = jnp.max(jnp.where(bc8 > 0, jnp.arange(E), 0)).astype(jnp.int32)
    block_expert = jnp.where(barange < num_active, owner, last_e).astype(
        jnp.int32)
    s0 = slot_out[:, 0]
    s1 = slot_out[:, 1]

    TPC = 8
    xd = _sc_dispatch(flat, s0.reshape(NW, N // (NW * TPC), TPC),
                      s1.reshape(NW, N // (NW * TPC), TPC), NSLOT)

    def grouped_body(be_ref, na_ref, xd_ref, wg_ref, wu_ref, wd_ref, yw_ref):
        b = pl.program_id(0)

        @pl.when(b < na_ref[0])
        def _():
            bf = jnp.bfloat16
            xb = xd_ref[...].astype(bf)
            g = jnp.dot(xb, wg_ref[0].astype(bf), preferred_element_type=f32)
            u = jnp.dot(xb, wu_ref[0].astype(bf), preferred_element_type=f32)
            h = (g * jax.nn.sigmoid(g) * u).astype(bf)
            yw_ref[...] = jnp.dot(h, wd_ref[0].astype(bf),
                                  preferred_element_type=f32)

    grid_spec = pltpu.PrefetchScalarGridSpec(
        num_scalar_prefetch=2,
        grid=(NB,),
        in_specs=[
            pl.BlockSpec((M, D),
                         lambda b, be, na: (jnp.minimum(b, na[0] - 1), 0)),
            pl.BlockSpec((1, D, H), lambda b, be, na: (be[b], 0, 0)),
            pl.BlockSpec((1, D, H), lambda b, be, na: (be[b], 0, 0)),
            pl.BlockSpec((1, H, D), lambda b, be, na: (be[b], 0, 0)),
        ],
        out_specs=pl.BlockSpec(
            (M, D), lambda b, be, na: (jnp.minimum(b, na[0] - 1), 0)),
    )
    yw = pl.pallas_call(
        grouped_body,
        grid_spec=grid_spec,
        out_shape=jax.ShapeDtypeStruct((NSLOT, D), f32),
    )(block_expert, na_arr, xd, Wg, Wu, Wd)

    pos_all = jnp.concatenate([s0, s1]).astype(jnp.int32)
    yg = _sc_gather(yw, pos_all.reshape(NW, (N * K) // (NW * 32), 32),
                    N * K, nbuf=1)

    def combine_body(s_ref, y0_ref, y1_ref, w_ref, o_ref):
        o_ref[...] = (s_ref[...] + y0_ref[...] * w_ref[:, 0:1]
                      + y1_ref[...] * w_ref[:, 1:2])

    nblk = N // TB
    out = pl.pallas_call(
        combine_body,
        grid=(nblk,),
        in_specs=[
            pl.BlockSpec((TB, D), lambda i: (i, 0)),
            pl.BlockSpec((TB, D), lambda i: (i, 0)),
            pl.BlockSpec((TB, D), lambda i, _n=nblk: (i + _n, 0)),
            pl.BlockSpec((TB, 128), lambda i: (i, 0)),
        ],
        out_specs=pl.BlockSpec((TB, D), lambda i: (i, 0)),
        out_shape=jax.ShapeDtypeStruct((N, D), f32),
    )(shared_out, yg, yg, w_out)

    return out.reshape(Bb, Tt, D)


# speedup vs baseline: 2.4770x; 1.0044x over previous
import functools

import jax
import jax.numpy as jnp
from jax import lax
from jax.experimental import pallas as pl
from jax.experimental.pallas import tpu as pltpu
from jax.experimental.pallas import tpu_sc as plsc

_NEG = -1e30


def _sc_gather(table, idx3, n_out_rows, nbuf=2):
    nw, chunks, chunk = idx3.shape
    d = table.shape[1]
    rows_per_w = chunks * chunk
    mesh = plsc.VectorSubcoreMesh(core_axis_name="c", subcore_axis_name="s")

    @functools.partial(
        pl.kernel,
        mesh=mesh,
        out_type=jax.ShapeDtypeStruct((n_out_rows, d), table.dtype),
        scratch_types=[
            pltpu.VMEM((chunks, chunk), jnp.int32),
            pltpu.VMEM((chunk, d), table.dtype),
            pltpu.VMEM((chunk, d), table.dtype),
            pltpu.SemaphoreType.DMA,
            pltpu.SemaphoreType.DMA,
        ],
    )
    def k(table_hbm, idx_hbm, out_hbm, idx_v, rows_v0, rows_v1, sem0, sem1):
        wid = lax.axis_index("s") * 2 + lax.axis_index("c")
        pltpu.sync_copy(idx_hbm.at[wid], idx_v)
        base = wid * rows_per_w
        if nbuf == 1:
            for c in range(chunks):
                pltpu.async_copy(
                    table_hbm.at[idx_v.at[c]], rows_v0, sem0).wait()
                pltpu.sync_copy(rows_v0,
                                out_hbm.at[pl.ds(base + c * chunk, chunk)])
            return
        bufs = (rows_v0, rows_v1)
        sems = (sem0, sem1)
        pend = [pltpu.async_copy(table_hbm.at[idx_v.at[0]], rows_v0, sem0),
                None]
        for c in range(chunks):
            if c + 1 < chunks:
                nb = (c + 1) % 2
                pend[nb] = pltpu.async_copy(
                    table_hbm.at[idx_v.at[c + 1]], bufs[nb], sems[nb])
            pend[c % 2].wait()
            pltpu.sync_copy(bufs[c % 2],
                            out_hbm.at[pl.ds(base + c * chunk, chunk)])

    return k(table, idx3)


def _sc_dispatch(x, s0_3, s1_3, n_out_rows):
    nw, chunks, tpc = s0_3.shape
    d = x.shape[1]
    tpw = chunks * tpc
    mesh = plsc.VectorSubcoreMesh(core_axis_name="c", subcore_axis_name="s")

    @functools.partial(
        pl.kernel,
        mesh=mesh,
        out_type=jax.ShapeDtypeStruct((n_out_rows, d), x.dtype),
        scratch_types=[
            pltpu.VMEM((chunks, tpc), jnp.int32),
            pltpu.VMEM((chunks, tpc), jnp.int32),
            pltpu.VMEM((tpc, d), x.dtype),
            pltpu.VMEM((tpc, d), x.dtype),
            pltpu.SemaphoreType.DMA,
            pltpu.SemaphoreType.DMA,
            pltpu.SemaphoreType.DMA,
            pltpu.SemaphoreType.DMA,
        ],
    )
    def k(x_hbm, s0_hbm, s1_hbm, out_hbm, s0_v, s1_v, buf0, buf1,
          sg0, sg1, ss0, ss1):
        wid = lax.axis_index("s") * 2 + lax.axis_index("c")
        pltpu.sync_copy(s0_hbm.at[wid], s0_v)
        pltpu.sync_copy(s1_hbm.at[wid], s1_v)
        base = wid * tpw
        bufs = (buf0, buf1)
        sg = (sg0, sg1)
        ss = (ss0, ss1)
        pend_g = [pltpu.async_copy(x_hbm.at[pl.ds(base, tpc)], buf0, sg0),
                  None]
        pend_s = [None, None]
        for c in range(chunks):
            if c + 1 < chunks:
                nb = (c + 1) % 2
                if pend_s[nb] is not None:
                    for h in pend_s[nb]:
                        h.wait()
                    pend_s[nb] = None
                pend_g[nb] = pltpu.async_copy(
                    x_hbm.at[pl.ds(base + (c + 1) * tpc, tpc)], bufs[nb],
                    sg[nb])
            cb = c % 2
            pend_g[cb].wait()
            pend_s[cb] = (
                pltpu.async_copy(bufs[cb], out_hbm.at[s0_v.at[c]], ss[cb]),
                pltpu.async_copy(bufs[cb], out_hbm.at[s1_v.at[c]], ss[cb]),
            )
        for p in pend_s:
            if p is not None:
                for h in p:
                    h.wait()

    return k(x, s0_3, s1_3)


def kernel(x, Wr, router_bias, Wg, Wu, Wd, Wsg, Wsu, Wsd):
    Bb, Tt, D = x.shape
    N = Bb * Tt
    E, _, H = Wg.shape
    S, _, HS = Wsg.shape
    K = 2
    M = 128
    NB = (K * N) // M + E
    NSLOT = NB * M
    TB = 256
    NW = 32

    flat = x.reshape(N, D)
    f32 = jnp.float32

    wrt = jnp.zeros((D, 128), f32).at[:, :E].set(Wr.T)
    biasp = jnp.full((1, 128), _NEG, f32).at[0, :E].set(router_bias)

    def router_body(x_ref, wrt_ref, bias_ref, idx_ref, w_ref, m_ref):
        xb = x_ref[...]
        logits = jnp.dot(xb, wrt_ref[...], preferred_element_type=f32)
        lane = lax.broadcasted_iota(jnp.int32, logits.shape, 1)
        valid = lane < E
        lm = jnp.where(valid, logits, _NEG)
        mx = jnp.max(lm, axis=1, keepdims=True)
        p = jnp.exp(lm - mx)
        scores = p / jnp.sum(p, axis=1, keepdims=True)
        sel = lm + bias_ref[...]
        m0 = jnp.max(sel, axis=1, keepdims=True)
        i0 = jnp.min(jnp.where(sel == m0, lane, 1000), axis=1, keepdims=True)
        pick0 = lane == i0
        sel2 = jnp.where(pick0, _NEG, sel)
        m1 = jnp.max(sel2, axis=1, keepdims=True)
        i1 = jnp.min(jnp.where(sel2 == m1, lane, 1000), axis=1, keepdims=True)
        pick1 = lane == i1
        w0 = jnp.sum(jnp.where(pick0, scores, 0.0), axis=1, keepdims=True)
        w1 = jnp.sum(jnp.where(pick1, scores, 0.0), axis=1, keepdims=True)
        tot = w0 + w1
        idx_ref[...] = jnp.where(lane == 0, i0, jnp.where(lane == 1, i1, 0))
        w_ref[...] = jnp.where(
            lane == 0, w0 / tot, jnp.where(lane == 1, w1 / tot, 0.0))
        m_ref[...] = jnp.where(pick0 | pick1, 1.0, 0.0)

    idx_out, w_out, m_out = pl.pallas_call(
        router_body,
        grid=(N // TB,),
        in_specs=[
            pl.BlockSpec((TB, D), lambda i: (i, 0)),
            pl.BlockSpec((D, 128), lambda i: (0, 0)),
            pl.BlockSpec((1, 128), lambda i: (0, 0)),
        ],
        out_specs=[
            pl.BlockSpec((TB, 128), lambda i: (i, 0)),
            pl.BlockSpec((TB, 128), lambda i: (i, 0)),
            pl.BlockSpec((TB, 128), lambda i: (i, 0)),
        ],
        out_shape=[
            jax.ShapeDtypeStruct((N, 128), jnp.int32),
            jax.ShapeDtypeStruct((N, 128), f32),
            jax.ShapeDtypeStruct((N, 128), f32),
        ],
    )(flat, wrt, biasp)

    TBS = 1024
    CH = 256
    HC = HS // CH
    NJ = S * HC

    def shared_body(x_ref, wsg_ref, wsu_ref, wsd_ref, o_ref):
        j = pl.program_id(1)
        bf = jnp.bfloat16
        xb = x_ref[...].astype(bf)
        g = jnp.dot(xb, wsg_ref[0].astype(bf), preferred_element_type=f32)
        u = jnp.dot(xb, wsu_ref[0].astype(bf), preferred_element_type=f32)
        h = (g * jax.nn.sigmoid(g) * u).astype(bf)
        part = jnp.dot(h, wsd_ref[0].astype(bf), preferred_element_type=f32)

        @pl.when(j == 0)
        def _():
            o_ref[...] = part

        @pl.when(j > 0)
        def _():
            o_ref[...] += part

    shared_out = pl.pallas_call(
        shared_body,
        grid=(N // TBS, NJ),
        in_specs=[
            pl.BlockSpec((TBS, D), lambda i, j: (i, 0)),
            pl.BlockSpec((1, D, CH), lambda i, j, _h=HC: (j // _h, 0, j % _h)),
            pl.BlockSpec((1, D, CH), lambda i, j, _h=HC: (j // _h, 0, j % _h)),
            pl.BlockSpec((1, CH, D), lambda i, j, _h=HC: (j // _h, j % _h, 0)),
        ],
        out_specs=pl.BlockSpec((TBS, D), lambda i, j: (i, 0)),
        out_shape=jax.ShapeDtypeStruct((N, D), f32),
    )(flat, Wsg, Wsu, Wsd)


    tri = jnp.triu(jnp.ones((128, 128), f32), k=1)

    def dispatch_body(m_ref, idx_ref, tri_ref, slot_ref, meta_ref):
        m = m_ref[...]
        lane = lax.broadcasted_iota(jnp.int32, m.shape, 1)
        inc = m
        sh = 1
        while sh < N:
            inc = inc + jnp.concatenate(
                [jnp.zeros((sh, 128), f32), inc[:N - sh]], axis=0)
            sh *= 2
        excl = inc - m
        counts = inc[N - 1:N, :]
        bc = jnp.floor((counts + (M - 1)) * (1.0 / M))
        blk_start = jnp.dot(bc, tri_ref[...], preferred_element_type=f32)
        tot = excl + blk_start * M
        i0 = idx_ref[:, 0:1]
        i1 = idx_ref[:, 1:2]
        s0 = jnp.sum(jnp.where(lane == i0, tot, 0.0), axis=1, keepdims=True)
        s1 = jnp.sum(jnp.where(lane == i1, tot, 0.0), axis=1, keepdims=True)
        slot_ref[...] = jnp.where(
            lane == 0, s0, jnp.where(lane == 1, s1, 0.0)).astype(jnp.int32)
        srow = lax.broadcasted_iota(jnp.int32, (8, 128), 0)
        meta_ref[...] = jnp.where(
            srow == 0, jnp.broadcast_to(blk_start, (8, 128)),
            jnp.where(srow == 1, jnp.broadcast_to(bc, (8, 128)),
                      0.0)).astype(jnp.int32)

    slot_out, meta = pl.pallas_call(
        dispatch_body,
        grid=(1,),
        in_specs=[
            pl.BlockSpec((N, 128), lambda i: (0, 0)),
            pl.BlockSpec((N, 128), lambda i: (0, 0)),
            pl.BlockSpec((128, 128), lambda i: (0, 0)),
        ],
        out_specs=[
            pl.BlockSpec((N, 128), lambda i: (0, 0)),
            pl.BlockSpec((8, 128), lambda i: (0, 0)),
        ],
        out_shape=[
            jax.ShapeDtypeStruct((N, 128), jnp.int32),
            jax.ShapeDtypeStruct((8, 128), jnp.int32),
        ],
    )(m_out, idx_out, tri)

    bs8 = meta[0, :E]
    bc8 = meta[1, :E]
    num_active = jnp.sum(bc8).astype(jnp.int32)
    na_arr = num_active.reshape(1)
    barange = jnp.arange(NB, dtype=jnp.int32)
    owner = jnp.sum(
        (bs8[None, :] <= barange[:, None]).astype(jnp.int32), axis=1) - 1
    last_e = jnp.max(jnp.where(bc8 > 0, jnp.arange(E), 0)).astype(jnp.int32)
    block_expert = jnp.where(barange < num_active, owner, last_e).astype(
        jnp.int32)
    s0 = slot_out[:, 0]
    s1 = slot_out[:, 1]

    TPC = 16
    xd = _sc_dispatch(flat, s0.reshape(NW, N // (NW * TPC), TPC),
                      s1.reshape(NW, N // (NW * TPC), TPC), NSLOT)

    def grouped_body(be_ref, na_ref, xd_ref, wg_ref, wu_ref, wd_ref, yw_ref):
        b = pl.program_id(0)

        @pl.when(b < na_ref[0])
        def _():
            bf = jnp.bfloat16
            xb = xd_ref[...].astype(bf)
            g = jnp.dot(xb, wg_ref[0].astype(bf), preferred_element_type=f32)
            u = jnp.dot(xb, wu_ref[0].astype(bf), preferred_element_type=f32)
            h = (g * jax.nn.sigmoid(g) * u).astype(bf)
            yw_ref[...] = jnp.dot(h, wd_ref[0].astype(bf),
                                  preferred_element_type=f32)

    grid_spec = pltpu.PrefetchScalarGridSpec(
        num_scalar_prefetch=2,
        grid=(NB,),
        in_specs=[
            pl.BlockSpec((M, D),
                         lambda b, be, na: (jnp.minimum(b, na[0] - 1), 0)),
            pl.BlockSpec((1, D, H), lambda b, be, na: (be[b], 0, 0)),
            pl.BlockSpec((1, D, H), lambda b, be, na: (be[b], 0, 0)),
            pl.BlockSpec((1, H, D), lambda b, be, na: (be[b], 0, 0)),
        ],
        out_specs=pl.BlockSpec(
            (M, D), lambda b, be, na: (jnp.minimum(b, na[0] - 1), 0)),
    )
    yw = pl.pallas_call(
        grouped_body,
        grid_spec=grid_spec,
        out_shape=jax.ShapeDtypeStruct((NSLOT, D), f32),
    )(block_expert, na_arr, xd, Wg, Wu, Wd)

    pos_all = jnp.concatenate([s0, s1]).astype(jnp.int32)
    yg = _sc_gather(yw, pos_all.reshape(NW, (N * K) // (NW * 32), 32),
                    N * K, nbuf=1)

    def combine_body(s_ref, y0_ref, y1_ref, w_ref, o_ref):
        o_ref[...] = (s_ref[...] + y0_ref[...] * w_ref[:, 0:1]
                      + y1_ref[...] * w_ref[:, 1:2])

    nblk = N // TB
    out = pl.pallas_call(
        combine_body,
        grid=(nblk,),
        in_specs=[
            pl.BlockSpec((TB, D), lambda i: (i, 0)),
            pl.BlockSpec((TB, D), lambda i: (i, 0)),
            pl.BlockSpec((TB, D), lambda i, _n=nblk: (i + _n, 0)),
            pl.BlockSpec((TB, 128), lambda i: (i, 0)),
        ],
        out_specs=pl.BlockSpec((TB, D), lambda i: (i, 0)),
        out_shape=jax.ShapeDtypeStruct((N, D), f32),
    )(shared_out, yg, yg, w_out)

    return out.reshape(Bb, Tt, D)
